# baseline (speedup 1.0000x reference)
import functools

import jax
import jax.numpy as jnp
from jax import lax
from jax.experimental import pallas as pl
from jax.experimental.pallas import tpu as pltpu
from jax.experimental.pallas import tpu_sc as plsc

D = 1024
N_EXP = 8
HID = 4 * D
S = 2048
NA = 2 * S
T_BLK = 128
NT = 40
NP = NT * T_BLK
H_BLK = 1024
N_H = HID // H_BLK


def _route_body(x_ref, wg_ref, bg_ref, dest_ref, val_ref, te_ref):
    lt = jnp.dot(x_ref[...], wg_ref[...],
                 preferred_element_type=jnp.float32) + bg_ref[...]
    m = jnp.max(lt, axis=-1, keepdims=True)
    ex = jnp.exp(lt - m)
    p = ex / jnp.sum(ex, axis=-1, keepdims=True)

    col = lax.broadcasted_iota(jnp.int32, (S, N_EXP), 1)
    big = jnp.int32(N_EXP)
    p1 = jnp.max(p, axis=-1, keepdims=True)
    i1 = jnp.min(jnp.where(p == p1, col, big), axis=-1, keepdims=True)
    masked = jnp.where(col == i1, -jnp.inf, p)
    p2 = jnp.max(masked, axis=-1, keepdims=True)
    i2 = jnp.min(jnp.where(masked == p2, col, big), axis=-1, keepdims=True)

    h1 = (col == i1).astype(jnp.float32)
    h2 = (col == i2).astype(jnp.float32)

    r = lax.broadcasted_iota(jnp.int32, (S, S), 0)
    c = lax.broadcasted_iota(jnp.int32, (S, S), 1)
    lt_mat = (r > c).astype(jnp.float32)
    ce = jnp.dot(lt_mat, h1 + h2, preferred_element_type=jnp.float32)

    rank0 = jnp.sum(h1 * ce, axis=-1, keepdims=True)
    rank1 = jnp.sum(h2 * (ce + h1), axis=-1, keepdims=True)

    cnt = jnp.sum(h1 + h2, axis=0, keepdims=True)
    pc = jnp.floor((cnt + (T_BLK - 1)) / T_BLK) * T_BLK
    er = lax.broadcasted_iota(jnp.int32, (N_EXP, N_EXP), 0)
    ec = lax.broadcasted_iota(jnp.int32, (N_EXP, N_EXP), 1)
    m8 = (er < ec).astype(jnp.float32)
    off = jnp.dot(pc, m8, preferred_element_type=jnp.float32)
    end = off + pc

    off0 = jnp.sum(h1 * off, axis=-1, keepdims=True)
    off1 = jnp.sum(h2 * off, axis=-1, keepdims=True)
    dest_ref[...] = jnp.concatenate(
        [off0 + rank0, off1 + rank1], axis=1).astype(jnp.int32)
    val_ref[...] = jnp.concatenate([p1, p2], axis=1)

    thr = (lax.broadcasted_iota(jnp.int32, (T_BLK, N_EXP), 0)
           * T_BLK).astype(jnp.float32)
    ge = (thr >= end).astype(jnp.float32)
    te = jnp.minimum(jnp.sum(ge, axis=-1, keepdims=True),
                     float(N_EXP - 1))
    te_ref[...] = te.astype(jnp.int32)


def _route(xf, Wg, bg):
    return pl.pallas_call(
        _route_body,
        out_shape=[
            jax.ShapeDtypeStruct((S, 2), jnp.int32),
            jax.ShapeDtypeStruct((S, 2), jnp.float32),
            jax.ShapeDtypeStruct((T_BLK, 1), jnp.int32),
        ],
    )(xf, Wg, bg)


_SC_MESH = dict(core_axis_name="c", subcore_axis_name="s")


def _scatter_body(dest_hbm, val_hbm, tok_hbm, g_hbm, dvm, vvm, tvm, gvm):
    wid = lax.axis_index("s") * 2 + lax.axis_index("c")

    @pl.when(wid == 0)
    def _():
        pltpu.sync_copy(dest_hbm, dvm)
        pltpu.sync_copy(val_hbm, vvm)

        def zero_body(i, _):
            tvm[pl.ds(i * 16, 16)] = jnp.zeros((16,), jnp.int32)
            gvm[pl.ds(i * 16, 16)] = jnp.zeros((16,), jnp.float32)
            return _
        lax.fori_loop(0, NP // 16, zero_body, 0)

        lanes = lax.iota(jnp.int32, 16)

        def sc_body(i, _):
            base = i * 16
            jj = lanes + base
            tok = lax.shift_right_logical(jj, 1)
            d = dvm[pl.ds(base, 16)]
            plsc.store_scatter(tvm, [d], tok)
            plsc.store_scatter(gvm, [d], vvm[pl.ds(base, 16)])
            return _
        lax.fori_loop(0, NA // 16, sc_body, 0)

        pltpu.sync_copy(tvm, tok_hbm)
        pltpu.sync_copy(gvm, g_hbm)


def _scatter_slots(dest_flat, val_flat):
    f = pl.kernel(
        _scatter_body, mesh=plsc.VectorSubcoreMesh(**_SC_MESH),
        compiler_params=pltpu.CompilerParams(needs_layout_passes=False),
        out_type=[
            jax.ShapeDtypeStruct((NP,), jnp.int32),
            jax.ShapeDtypeStruct((NP,), jnp.float32),
        ],
        scratch_types=[
            pltpu.VMEM((NA,), jnp.int32),
            pltpu.VMEM((NA,), jnp.float32),
            pltpu.VMEM((NP,), jnp.int32),
            pltpu.VMEM((NP,), jnp.float32),
        ],
    )
    return f(dest_flat, val_flat)


_GROWS = NP // 32
_GCH = 2
_GCROWS = _GROWS // _GCH


def _gather_body(tok_hbm, x_hbm, xs_hbm, idxv, rowsv, sem):
    wid = lax.axis_index("s") * 2 + lax.axis_index("c")
    base = wid * _GROWS
    for ch in range(_GCH):
        lo = base + ch * _GCROWS
        pltpu.sync_copy(tok_hbm.at[pl.ds(lo, _GCROWS)], idxv)
        pltpu.async_copy(x_hbm.at[idxv], rowsv, sem).wait()
        pltpu.sync_copy(rowsv, xs_hbm.at[pl.ds(lo, _GCROWS)])


def _gather_rows(tok_s, xf):
    f = pl.kernel(
        _gather_body, mesh=plsc.VectorSubcoreMesh(**_SC_MESH),
        out_type=jax.ShapeDtypeStruct((NP, D), jnp.float32),
        scratch_types=[
            pltpu.VMEM((_GCROWS,), jnp.int32),
            pltpu.VMEM((_GCROWS, D), jnp.float32),
            pltpu.SemaphoreType.DMA,
        ],
    )
    return f(tok_s, xf)


def _ffn_body(te_ref, xs_ref, w1_ref, b1_ref, w2_ref, b2_ref, g_ref,
              eo_ref, acc_ref):
    h = pl.program_id(0)
    i = pl.program_id(1)
    rows = pl.ds(i * T_BLK, T_BLK)

    xb = xs_ref[...].astype(jnp.bfloat16)
    hb = jnp.dot(xb, w1_ref[0], preferred_element_type=jnp.float32)
    hb = hb + b1_ref[0]
    hb = hb * 0.5 * (1.0 + lax.erf(hb * 0.7071067811865476))
    contrib = jnp.dot(hb.astype(jnp.bfloat16), w2_ref[0],
                      preferred_element_type=jnp.float32)

    @pl.when(h == 0)
    def _():
        acc_ref[rows, :] = contrib + b2_ref[0]

    @pl.when(h != 0)
    def _():
        acc_ref[rows, :] += contrib

    eo_ref[...] = g_ref[0] * acc_ref[rows, :]


def _ffn(tile_e, xs, W1, b1r, W2, b2r, g3):
    grid_spec = pltpu.PrefetchScalarGridSpec(
        num_scalar_prefetch=1,
        grid=(N_H, NT),
        in_specs=[
            pl.BlockSpec((T_BLK, D), lambda h, i, te: (i, 0)),
            pl.BlockSpec((1, D, H_BLK), lambda h, i, te: (te[i], 0, h)),
            pl.BlockSpec((1, 1, H_BLK), lambda h, i, te: (te[i], 0, h)),
            pl.BlockSpec((1, H_BLK, D), lambda h, i, te: (te[i], h, 0)),
            pl.BlockSpec((1, 1, D), lambda h, i, te: (te[i], 0, 0)),
            pl.BlockSpec((1, T_BLK, 1), lambda h, i, te: (i, 0, 0)),
        ],
        out_specs=pl.BlockSpec((T_BLK, D), lambda h, i, te: (i, 0)),
        scratch_shapes=[pltpu.VMEM((NP, D), jnp.float32)],
    )
    return pl.pallas_call(
        _ffn_body,
        grid_spec=grid_spec,
        out_shape=jax.ShapeDtypeStruct((NP, D), jnp.float32),
    )(tile_e, xs, W1, b1r, W2, b2r, g3)


_CTOK = S // 32
_CCH = 2
_CCTOK = _CTOK // _CCH


def _combine_body(dest_hbm, eo_hbm, out_hbm, idxv, rowsv, outv, sem):
    wid = lax.axis_index("s") * 2 + lax.axis_index("c")
    for ch in range(_CCH):
        t0 = wid * _CTOK + ch * _CCTOK
        pltpu.sync_copy(dest_hbm.at[pl.ds(2 * t0, 2 * _CCTOK)], idxv)
        pltpu.async_copy(eo_hbm.at[idxv], rowsv, sem).wait()

        def tok_body(i, _):
            for v in range(D // 16):
                sl = pl.ds(v * 16, 16)
                outv[i, sl] = rowsv[2 * i, sl] + rowsv[2 * i + 1, sl]
            return _
        lax.fori_loop(0, _CCTOK, tok_body, 0)
        pltpu.sync_copy(outv, out_hbm.at[pl.ds(t0, _CCTOK)])


def _combine(dest_flat, eo):
    f = pl.kernel(
        _combine_body, mesh=plsc.VectorSubcoreMesh(**_SC_MESH),
        out_type=jax.ShapeDtypeStruct((S, D), jnp.float32),
        scratch_types=[
            pltpu.VMEM((2 * _CCTOK,), jnp.int32),
            pltpu.VMEM((2 * _CCTOK, D), jnp.float32),
            pltpu.VMEM((_CCTOK, D), jnp.float32),
            pltpu.SemaphoreType.DMA,
        ],
    )
    return f(dest_flat, eo)


def kernel(x, Wg, bg, W1, b1, W2, b2):
    b, s, d = x.shape
    xf = x.reshape(s, d)

    dest2, val2, te = _route(xf, Wg, bg)
    dest_flat = dest2.reshape(NA)
    val_flat = val2.reshape(NA)

    tok_s, g_s = _scatter_slots(dest_flat, val_flat)
    xs = _gather_rows(tok_s, xf)

    eo = _ffn(te.reshape(T_BLK), xs, W1.astype(jnp.bfloat16),
              b1.reshape(N_EXP, 1, HID), W2.astype(jnp.bfloat16),
              b2.reshape(N_EXP, 1, D), g_s.reshape(NT, T_BLK, 1))

    out = _combine(dest_flat, eo)
    return out.reshape(b, s, d)


# speedup vs baseline: 1.1999x; 1.1999x over previous
import functools

import jax
import jax.numpy as jnp
from jax import lax
from jax.experimental import pallas as pl
from jax.experimental.pallas import tpu as pltpu
from jax.experimental.pallas import tpu_sc as plsc

D = 1024
N_EXP = 8
HID = 4 * D
S = 2048
NA = 2 * S
T_BLK = 128
NT = 40
NP = NT * T_BLK
H_BLK = 1024
N_H = HID // H_BLK


def _route_body(x_ref, wg_ref, bg_ref, dest_ref, val_ref, te_ref):
    lt = jnp.dot(x_ref[...], wg_ref[...],
                 preferred_element_type=jnp.float32) + bg_ref[...]
    m = jnp.max(lt, axis=-1, keepdims=True)
    ex = jnp.exp(lt - m)
    p = ex / jnp.sum(ex, axis=-1, keepdims=True)

    col = lax.broadcasted_iota(jnp.int32, (S, N_EXP), 1)
    big = jnp.int32(N_EXP)
    p1 = jnp.max(p, axis=-1, keepdims=True)
    i1 = jnp.min(jnp.where(p == p1, col, big), axis=-1, keepdims=True)
    masked = jnp.where(col == i1, -jnp.inf, p)
    p2 = jnp.max(masked, axis=-1, keepdims=True)
    i2 = jnp.min(jnp.where(masked == p2, col, big), axis=-1, keepdims=True)

    h1 = (col == i1).astype(jnp.float32)
    h2 = (col == i2).astype(jnp.float32)

    r = lax.broadcasted_iota(jnp.int32, (S, S), 0)
    c = lax.broadcasted_iota(jnp.int32, (S, S), 1)
    lt_mat = (r > c).astype(jnp.float32)
    ce = jnp.dot(lt_mat, h1 + h2, preferred_element_type=jnp.float32)

    rank0 = jnp.sum(h1 * ce, axis=-1, keepdims=True)
    rank1 = jnp.sum(h2 * (ce + h1), axis=-1, keepdims=True)

    cnt = jnp.sum(h1 + h2, axis=0, keepdims=True)
    pc = jnp.floor((cnt + (T_BLK - 1)) / T_BLK) * T_BLK
    er = lax.broadcasted_iota(jnp.int32, (N_EXP, N_EXP), 0)
    ec = lax.broadcasted_iota(jnp.int32, (N_EXP, N_EXP), 1)
    m8 = (er < ec).astype(jnp.float32)
    off = jnp.dot(pc, m8, preferred_element_type=jnp.float32)
    end = off + pc

    off0 = jnp.sum(h1 * off, axis=-1, keepdims=True)
    off1 = jnp.sum(h2 * off, axis=-1, keepdims=True)
    dest_ref[...] = jnp.concatenate(
        [off0 + rank0, off1 + rank1], axis=1).astype(jnp.int32)
    val_ref[...] = jnp.concatenate([p1, p2], axis=1)

    thr = (lax.broadcasted_iota(jnp.int32, (T_BLK, N_EXP), 0)
           * T_BLK).astype(jnp.float32)
    ge = (thr >= end).astype(jnp.float32)
    te = jnp.minimum(jnp.sum(ge, axis=-1, keepdims=True),
                     float(N_EXP - 1))
    te_ref[...] = te.astype(jnp.int32)


def _route(xf, Wg, bg):
    return pl.pallas_call(
        _route_body,
        out_shape=[
            jax.ShapeDtypeStruct((S, 2), jnp.int32),
            jax.ShapeDtypeStruct((S, 2), jnp.float32),
            jax.ShapeDtypeStruct((T_BLK, 1), jnp.int32),
        ],
    )(xf, Wg, bg)


_SC_MESH = dict(core_axis_name="c", subcore_axis_name="s")


def _scatter_body(dest_hbm, val_hbm, tok_hbm, g_hbm, dvm, vvm, tvm, gvm):
    wid = lax.axis_index("s") * 2 + lax.axis_index("c")

    @pl.when(wid == 0)
    def _():
        pltpu.sync_copy(dest_hbm, dvm)
        pltpu.sync_copy(val_hbm, vvm)

        def zero_body(i, _):
            tvm[pl.ds(i * 16, 16)] = jnp.zeros((16,), jnp.int32)
            gvm[pl.ds(i * 16, 16)] = jnp.zeros((16,), jnp.float32)
            return _
        lax.fori_loop(0, NP // 16, zero_body, 0)

        lanes = lax.iota(jnp.int32, 16)

        def sc_body(i, _):
            base = i * 16
            jj = lanes + base
            tok = lax.shift_right_logical(jj, 1)
            d = dvm[pl.ds(base, 16)]
            plsc.store_scatter(tvm, [d], tok)
            plsc.store_scatter(gvm, [d], vvm[pl.ds(base, 16)])
            return _
        lax.fori_loop(0, NA // 16, sc_body, 0)

        pltpu.sync_copy(tvm, tok_hbm)
        pltpu.sync_copy(gvm, g_hbm)


def _scatter_slots(dest_flat, val_flat):
    f = pl.kernel(
        _scatter_body, mesh=plsc.VectorSubcoreMesh(**_SC_MESH),
        compiler_params=pltpu.CompilerParams(needs_layout_passes=False),
        out_type=[
            jax.ShapeDtypeStruct((NP,), jnp.int32),
            jax.ShapeDtypeStruct((NP,), jnp.float32),
        ],
        scratch_types=[
            pltpu.VMEM((NA,), jnp.int32),
            pltpu.VMEM((NA,), jnp.float32),
            pltpu.VMEM((NP,), jnp.int32),
            pltpu.VMEM((NP,), jnp.float32),
        ],
    )
    return f(dest_flat, val_flat)


_GROWS = NP // 32
_GCH = 2
_GCROWS = _GROWS // _GCH


def _gather_body(tok_hbm, x_hbm, xs_hbm, idxv, rowsv, sem):
    wid = lax.axis_index("s") * 2 + lax.axis_index("c")
    base = wid * _GROWS
    for ch in range(_GCH):
        lo = base + ch * _GCROWS
        pltpu.sync_copy(tok_hbm.at[pl.ds(lo, _GCROWS)], idxv)
        pltpu.async_copy(x_hbm.at[idxv], rowsv, sem).wait()
        pltpu.sync_copy(rowsv, xs_hbm.at[pl.ds(lo, _GCROWS)])


def _gather_rows(tok_s, xf):
    f = pl.kernel(
        _gather_body, mesh=plsc.VectorSubcoreMesh(**_SC_MESH),
        out_type=jax.ShapeDtypeStruct((NP, D), jnp.float32),
        scratch_types=[
            pltpu.VMEM((_GCROWS,), jnp.int32),
            pltpu.VMEM((_GCROWS, D), jnp.float32),
            pltpu.SemaphoreType.DMA,
        ],
    )
    return f(tok_s, xf)


def _ffn_body(te_ref, xs_ref, w1_ref, b1_ref, w2_ref, b2_ref, g_ref,
              eo_ref, acc_ref):
    h = pl.program_id(0)
    i = pl.program_id(1)
    rows = pl.ds(i * T_BLK, T_BLK)

    xb = xs_ref[...]
    hb = jnp.dot(xb, w1_ref[0], preferred_element_type=jnp.float32,
                 precision=lax.Precision.DEFAULT)
    hb = hb + b1_ref[0]
    hb = hb * 0.5 * (1.0 + lax.erf(hb * 0.7071067811865476))
    contrib = jnp.dot(hb, w2_ref[0], preferred_element_type=jnp.float32,
                      precision=lax.Precision.DEFAULT)

    @pl.when(h == 0)
    def _():
        acc_ref[rows, :] = contrib + b2_ref[0]

    @pl.when(h != 0)
    def _():
        acc_ref[rows, :] += contrib

    eo_ref[...] = g_ref[0] * acc_ref[rows, :]


def _ffn(tile_e, xs, W1, b1r, W2, b2r, g3):
    grid_spec = pltpu.PrefetchScalarGridSpec(
        num_scalar_prefetch=1,
        grid=(N_H, NT),
        in_specs=[
            pl.BlockSpec((T_BLK, D), lambda h, i, te: (i, 0)),
            pl.BlockSpec((1, D, H_BLK), lambda h, i, te: (te[i], 0, h)),
            pl.BlockSpec((1, 1, H_BLK), lambda h, i, te: (te[i], 0, h)),
            pl.BlockSpec((1, H_BLK, D), lambda h, i, te: (te[i], h, 0)),
            pl.BlockSpec((1, 1, D), lambda h, i, te: (te[i], 0, 0)),
            pl.BlockSpec((1, T_BLK, 1), lambda h, i, te: (i, 0, 0)),
        ],
        out_specs=pl.BlockSpec((T_BLK, D), lambda h, i, te: (i, 0)),
        scratch_shapes=[pltpu.VMEM((NP, D), jnp.float32)],
    )
    return pl.pallas_call(
        _ffn_body,
        grid_spec=grid_spec,
        out_shape=jax.ShapeDtypeStruct((NP, D), jnp.float32),
    )(tile_e, xs, W1, b1r, W2, b2r, g3)


_CTOK = S // 32
_CCH = 2
_CCTOK = _CTOK // _CCH


def _combine_body(dest_hbm, eo_hbm, out_hbm, idxv, rowsv, outv, sem):
    wid = lax.axis_index("s") * 2 + lax.axis_index("c")
    for ch in range(_CCH):
        t0 = wid * _CTOK + ch * _CCTOK
        pltpu.sync_copy(dest_hbm.at[pl.ds(2 * t0, 2 * _CCTOK)], idxv)
        pltpu.async_copy(eo_hbm.at[idxv], rowsv, sem).wait()

        def tok_body(i, _):
            for v in range(D // 16):
                sl = pl.ds(v * 16, 16)
                outv[i, sl] = rowsv[2 * i, sl] + rowsv[2 * i + 1, sl]
            return _
        lax.fori_loop(0, _CCTOK, tok_body, 0)
        pltpu.sync_copy(outv, out_hbm.at[pl.ds(t0, _CCTOK)])


def _combine(dest_flat, eo):
    f = pl.kernel(
        _combine_body, mesh=plsc.VectorSubcoreMesh(**_SC_MESH),
        out_type=jax.ShapeDtypeStruct((S, D), jnp.float32),
        scratch_types=[
            pltpu.VMEM((2 * _CCTOK,), jnp.int32),
            pltpu.VMEM((2 * _CCTOK, D), jnp.float32),
            pltpu.VMEM((_CCTOK, D), jnp.float32),
            pltpu.SemaphoreType.DMA,
        ],
    )
    return f(dest_flat, eo)


def kernel(x, Wg, bg, W1, b1, W2, b2):
    b, s, d = x.shape
    xf = x.reshape(s, d)

    dest2, val2, te = _route(xf, Wg, bg)
    dest_flat = dest2.reshape(NA)
    val_flat = val2.reshape(NA)

    tok_s, g_s = _scatter_slots(dest_flat, val_flat)
    xs = _gather_rows(tok_s, xf)

    eo = _ffn(te.reshape(T_BLK), xs, W1, b1.reshape(N_EXP, 1, HID),
              W2, b2.reshape(N_EXP, 1, D), g_s.reshape(NT, T_BLK, 1))

    out = _combine(dest_flat, eo)
    return out.reshape(b, s, d)


# speedup vs baseline: 1.4407x; 1.2006x over previous
import functools

import jax
import jax.numpy as jnp
from jax import lax
from jax.experimental import pallas as pl
from jax.experimental.pallas import tpu as pltpu
from jax.experimental.pallas import tpu_sc as plsc

D = 1024
N_EXP = 8
HID = 4 * D
S = 2048
NA = 2 * S
T_BLK = 128
NT = 40
NP = NT * T_BLK
H_BLK = 2048
N_H = HID // H_BLK


def _route_body(x_ref, wg_ref, bg_ref, dest_ref, val_ref, te_ref):
    lt = jnp.dot(x_ref[...], wg_ref[...],
                 preferred_element_type=jnp.float32) + bg_ref[...]
    m = jnp.max(lt, axis=-1, keepdims=True)
    ex = jnp.exp(lt - m)
    p = ex / jnp.sum(ex, axis=-1, keepdims=True)

    col = lax.broadcasted_iota(jnp.int32, (S, N_EXP), 1)
    big = jnp.int32(N_EXP)
    p1 = jnp.max(p, axis=-1, keepdims=True)
    i1 = jnp.min(jnp.where(p == p1, col, big), axis=-1, keepdims=True)
    masked = jnp.where(col == i1, -jnp.inf, p)
    p2 = jnp.max(masked, axis=-1, keepdims=True)
    i2 = jnp.min(jnp.where(masked == p2, col, big), axis=-1, keepdims=True)

    h1 = (col == i1).astype(jnp.float32)
    h2 = (col == i2).astype(jnp.float32)

    r = lax.broadcasted_iota(jnp.int32, (S, S), 0)
    c = lax.broadcasted_iota(jnp.int32, (S, S), 1)
    lt_mat = (r > c).astype(jnp.float32)
    ce = jnp.dot(lt_mat, h1 + h2, preferred_element_type=jnp.float32)

    rank0 = jnp.sum(h1 * ce, axis=-1, keepdims=True)
    rank1 = jnp.sum(h2 * (ce + h1), axis=-1, keepdims=True)

    cnt = jnp.sum(h1 + h2, axis=0, keepdims=True)
    pc = jnp.floor((cnt + (T_BLK - 1)) / T_BLK) * T_BLK
    er = lax.broadcasted_iota(jnp.int32, (N_EXP, N_EXP), 0)
    ec = lax.broadcasted_iota(jnp.int32, (N_EXP, N_EXP), 1)
    m8 = (er < ec).astype(jnp.float32)
    off = jnp.dot(pc, m8, preferred_element_type=jnp.float32)
    end = off + pc

    off0 = jnp.sum(h1 * off, axis=-1, keepdims=True)
    off1 = jnp.sum(h2 * off, axis=-1, keepdims=True)
    dest_ref[...] = jnp.concatenate(
        [off0 + rank0, off1 + rank1], axis=1).astype(jnp.int32)
    val_ref[...] = jnp.concatenate([p1, p2], axis=1)

    thr = (lax.broadcasted_iota(jnp.int32, (T_BLK, N_EXP), 0)
           * T_BLK).astype(jnp.float32)
    ge = (thr >= end).astype(jnp.float32)
    te = jnp.minimum(jnp.sum(ge, axis=-1, keepdims=True),
                     float(N_EXP - 1))
    te_ref[...] = te.astype(jnp.int32)


def _route(xf, Wg, bg):
    return pl.pallas_call(
        _route_body,
        out_shape=[
            jax.ShapeDtypeStruct((S, 2), jnp.int32),
            jax.ShapeDtypeStruct((S, 2), jnp.float32),
            jax.ShapeDtypeStruct((T_BLK, 1), jnp.int32),
        ],
    )(xf, Wg, bg)


_SC_MESH = dict(core_axis_name="c", subcore_axis_name="s")


def _scatter_body(dest_hbm, val_hbm, tok_hbm, g_hbm, dvm, vvm, tvm, gvm):
    wid = lax.axis_index("s") * 2 + lax.axis_index("c")

    @pl.when(wid == 0)
    def _():
        pltpu.sync_copy(dest_hbm, dvm)
        pltpu.sync_copy(val_hbm, vvm)

        def zero_body(i, _):
            tvm[pl.ds(i * 16, 16)] = jnp.zeros((16,), jnp.int32)
            gvm[pl.ds(i * 16, 16)] = jnp.zeros((16,), jnp.float32)
            return _
        lax.fori_loop(0, NP // 16, zero_body, 0)

        lanes = lax.iota(jnp.int32, 16)

        def sc_body(i, _):
            base = i * 16
            jj = lanes + base
            tok = lax.shift_right_logical(jj, 1)
            d = dvm[pl.ds(base, 16)]
            plsc.store_scatter(tvm, [d], tok)
            plsc.store_scatter(gvm, [d], vvm[pl.ds(base, 16)])
            return _
        lax.fori_loop(0, NA // 16, sc_body, 0)

        pltpu.sync_copy(tvm, tok_hbm)
        pltpu.sync_copy(gvm, g_hbm)


def _scatter_slots(dest_flat, val_flat):
    f = pl.kernel(
        _scatter_body, mesh=plsc.VectorSubcoreMesh(**_SC_MESH),
        compiler_params=pltpu.CompilerParams(needs_layout_passes=False),
        out_type=[
            jax.ShapeDtypeStruct((NP,), jnp.int32),
            jax.ShapeDtypeStruct((NP,), jnp.float32),
        ],
        scratch_types=[
            pltpu.VMEM((NA,), jnp.int32),
            pltpu.VMEM((NA,), jnp.float32),
            pltpu.VMEM((NP,), jnp.int32),
            pltpu.VMEM((NP,), jnp.float32),
        ],
    )
    return f(dest_flat, val_flat)


_GROWS = NP // 32
_GCH = 2
_GCROWS = _GROWS // _GCH


def _gather_body(tok_hbm, x_hbm, xs_hbm, idxv, rowsv, sem):
    wid = lax.axis_index("s") * 2 + lax.axis_index("c")
    base = wid * _GROWS
    for ch in range(_GCH):
        lo = base + ch * _GCROWS
        pltpu.sync_copy(tok_hbm.at[pl.ds(lo, _GCROWS)], idxv)
        pltpu.async_copy(x_hbm.at[idxv], rowsv, sem).wait()
        pltpu.sync_copy(rowsv, xs_hbm.at[pl.ds(lo, _GCROWS)])


def _gather_rows(tok_s, xf):
    f = pl.kernel(
        _gather_body, mesh=plsc.VectorSubcoreMesh(**_SC_MESH),
        out_type=jax.ShapeDtypeStruct((NP, D), jnp.float32),
        scratch_types=[
            pltpu.VMEM((_GCROWS,), jnp.int32),
            pltpu.VMEM((_GCROWS, D), jnp.float32),
            pltpu.SemaphoreType.DMA,
        ],
    )
    return f(tok_s, xf)


def _ffn_body(te_ref, xs_ref, w1_ref, b1_ref, w2_ref, b2_ref, g_ref,
              eo_ref, acc_ref):
    h = pl.program_id(0)
    i = pl.program_id(1)
    rows = pl.ds(i * T_BLK, T_BLK)

    xb = xs_ref[...]
    hb = jnp.dot(xb, w1_ref[0], preferred_element_type=jnp.float32)
    hb = hb + b1_ref[0]
    hb = hb * 0.5 * (1.0 + lax.erf(hb * 0.7071067811865476))
    contrib = jnp.dot(hb, w2_ref[0], preferred_element_type=jnp.float32)

    @pl.when(h == 0)
    def _():
        acc_ref[rows, :] = contrib + b2_ref[0]

    @pl.when(h != 0)
    def _():
        eo_ref[...] = g_ref[0] * (acc_ref[rows, :] + contrib)


def _ffn(tile_e, xs, W1, b1r, W2, b2r, g3):
    grid_spec = pltpu.PrefetchScalarGridSpec(
        num_scalar_prefetch=1,
        grid=(N_H, NT),
        in_specs=[
            pl.BlockSpec((T_BLK, D), lambda h, i, te: (i, 0)),
            pl.BlockSpec((1, D, H_BLK), lambda h, i, te: (te[i], 0, h)),
            pl.BlockSpec((1, 1, H_BLK), lambda h, i, te: (te[i], 0, h)),
            pl.BlockSpec((1, H_BLK, D), lambda h, i, te: (te[i], h, 0)),
            pl.BlockSpec((1, 1, D), lambda h, i, te: (te[i], 0, 0)),
            pl.BlockSpec((1, T_BLK, 1), lambda h, i, te: (i, 0, 0)),
        ],
        out_specs=pl.BlockSpec((T_BLK, D), lambda h, i, te: (i, 0)),
        scratch_shapes=[pltpu.VMEM((NP, D), jnp.float32)],
    )
    return pl.pallas_call(
        _ffn_body,
        grid_spec=grid_spec,
        out_shape=jax.ShapeDtypeStruct((NP, D), jnp.float32),
    )(tile_e, xs, W1, b1r, W2, b2r, g3)


_CTOK = S // 32
_CCH = 2
_CCTOK = _CTOK // _CCH


def _combine_body(dest_hbm, eo_hbm, out_hbm, idxv, rowsv, outv, sem):
    wid = lax.axis_index("s") * 2 + lax.axis_index("c")
    for ch in range(_CCH):
        t0 = wid * _CTOK + ch * _CCTOK
        pltpu.sync_copy(dest_hbm.at[pl.ds(2 * t0, 2 * _CCTOK)], idxv)
        pltpu.async_copy(eo_hbm.at[idxv], rowsv, sem).wait()

        def tok_body(i, _):
            for v in range(D // 16):
                sl = pl.ds(v * 16, 16)
                outv[i, sl] = rowsv[2 * i, sl] + rowsv[2 * i + 1, sl]
            return _
        lax.fori_loop(0, _CCTOK, tok_body, 0)
        pltpu.sync_copy(outv, out_hbm.at[pl.ds(t0, _CCTOK)])


def _combine(dest_flat, eo):
    f = pl.kernel(
        _combine_body, mesh=plsc.VectorSubcoreMesh(**_SC_MESH),
        out_type=jax.ShapeDtypeStruct((S, D), jnp.float32),
        scratch_types=[
            pltpu.VMEM((2 * _CCTOK,), jnp.int32),
            pltpu.VMEM((2 * _CCTOK, D), jnp.float32),
            pltpu.VMEM((_CCTOK, D), jnp.float32),
            pltpu.SemaphoreType.DMA,
        ],
    )
    return f(dest_flat, eo)


def kernel(x, Wg, bg, W1, b1, W2, b2):
    b, s, d = x.shape
    xf = x.reshape(s, d)

    dest2, val2, te = _route(xf, Wg, bg)
    dest_flat = dest2.reshape(NA)
    val_flat = val2.reshape(NA)

    tok_s, g_s = _scatter_slots(dest_flat, val_flat)
    xs = _gather_rows(tok_s, xf)

    eo = _ffn(te.reshape(T_BLK), xs, W1, b1.reshape(N_EXP, 1, HID),
              W2, b2.reshape(N_EXP, 1, D), g_s.reshape(NT, T_BLK, 1))

    out = _combine(dest_flat, eo)
    return out.reshape(b, s, d)


# speedup vs baseline: 1.4558x; 1.0105x over previous
import functools

import jax
import jax.numpy as jnp
from jax import lax
from jax.experimental import pallas as pl
from jax.experimental.pallas import tpu as pltpu
from jax.experimental.pallas import tpu_sc as plsc

D = 1024
N_EXP = 8
HID = 4 * D
S = 2048
NA = 2 * S
T_BLK = 128
NT = 40
NP = NT * T_BLK
H_BLK = 2048
N_H = HID // H_BLK


def _route_body(x_ref, wg_ref, bg_ref, dest_ref, val_ref, te_ref):
    lt = jnp.dot(x_ref[...], wg_ref[...],
                 preferred_element_type=jnp.float32) + bg_ref[...]
    m = jnp.max(lt, axis=-1, keepdims=True)
    ex = jnp.exp(lt - m)
    p = ex / jnp.sum(ex, axis=-1, keepdims=True)

    col = lax.broadcasted_iota(jnp.int32, (S, N_EXP), 1)
    big = jnp.int32(N_EXP)
    p1 = jnp.max(p, axis=-1, keepdims=True)
    i1 = jnp.min(jnp.where(p == p1, col, big), axis=-1, keepdims=True)
    masked = jnp.where(col == i1, -jnp.inf, p)
    p2 = jnp.max(masked, axis=-1, keepdims=True)
    i2 = jnp.min(jnp.where(masked == p2, col, big), axis=-1, keepdims=True)

    h1 = (col == i1).astype(jnp.float32)
    h2 = (col == i2).astype(jnp.float32)

    r = lax.broadcasted_iota(jnp.int32, (S, S), 0)
    c = lax.broadcasted_iota(jnp.int32, (S, S), 1)
    lt_mat = (r > c).astype(jnp.float32)
    ce = jnp.dot(lt_mat, h1 + h2, preferred_element_type=jnp.float32)

    rank0 = jnp.sum(h1 * ce, axis=-1, keepdims=True)
    rank1 = jnp.sum(h2 * (ce + h1), axis=-1, keepdims=True)

    cnt = jnp.sum(h1 + h2, axis=0, keepdims=True)
    pc = jnp.floor((cnt + (T_BLK - 1)) / T_BLK) * T_BLK
    er = lax.broadcasted_iota(jnp.int32, (N_EXP, N_EXP), 0)
    ec = lax.broadcasted_iota(jnp.int32, (N_EXP, N_EXP), 1)
    m8 = (er < ec).astype(jnp.float32)
    off = jnp.dot(pc, m8, preferred_element_type=jnp.float32)
    end = off + pc

    off0 = jnp.sum(h1 * off, axis=-1, keepdims=True)
    off1 = jnp.sum(h2 * off, axis=-1, keepdims=True)
    dest_ref[...] = jnp.concatenate(
        [off0 + rank0, off1 + rank1], axis=1).astype(jnp.int32)
    val_ref[...] = jnp.concatenate([p1, p2], axis=1)

    thr = (lax.broadcasted_iota(jnp.int32, (T_BLK, N_EXP), 0)
           * T_BLK).astype(jnp.float32)
    ge = (thr >= end).astype(jnp.float32)
    te = jnp.minimum(jnp.sum(ge, axis=-1, keepdims=True),
                     float(N_EXP - 1))
    te_ref[...] = te.astype(jnp.int32)


def _route(xf, Wg, bg):
    return pl.pallas_call(
        _route_body,
        out_shape=[
            jax.ShapeDtypeStruct((S, 2), jnp.int32),
            jax.ShapeDtypeStruct((S, 2), jnp.float32),
            jax.ShapeDtypeStruct((T_BLK, 1), jnp.int32),
        ],
    )(xf, Wg, bg)


_SC_MESH = dict(core_axis_name="c", subcore_axis_name="s")


def _scatter_body(dest_hbm, val_hbm, tok_hbm, g_hbm, dvm, vvm, tvm, gvm):
    wid = lax.axis_index("s") * 2 + lax.axis_index("c")

    @pl.when(wid == 0)
    def _():
        pltpu.sync_copy(dest_hbm, dvm)
        pltpu.sync_copy(val_hbm, vvm)

        def zero_body(i, _):
            tvm[pl.ds(i * 16, 16)] = jnp.zeros((16,), jnp.int32)
            gvm[pl.ds(i * 16, 16)] = jnp.zeros((16,), jnp.float32)
            return _
        lax.fori_loop(0, NP // 16, zero_body, 0)

        lanes = lax.iota(jnp.int32, 16)

        def sc_body(i, _):
            base = i * 16
            jj = lanes + base
            tok = lax.shift_right_logical(jj, 1)
            d = dvm[pl.ds(base, 16)]
            plsc.store_scatter(tvm, [d], tok)
            plsc.store_scatter(gvm, [d], vvm[pl.ds(base, 16)])
            return _
        lax.fori_loop(0, NA // 16, sc_body, 0)

        pltpu.sync_copy(tvm, tok_hbm)
        pltpu.sync_copy(gvm, g_hbm)


def _scatter_slots(dest_flat, val_flat):
    f = pl.kernel(
        _scatter_body, mesh=plsc.VectorSubcoreMesh(**_SC_MESH),
        compiler_params=pltpu.CompilerParams(needs_layout_passes=False),
        out_type=[
            jax.ShapeDtypeStruct((NP,), jnp.int32),
            jax.ShapeDtypeStruct((NP,), jnp.float32),
        ],
        scratch_types=[
            pltpu.VMEM((NA,), jnp.int32),
            pltpu.VMEM((NA,), jnp.float32),
            pltpu.VMEM((NP,), jnp.int32),
            pltpu.VMEM((NP,), jnp.float32),
        ],
    )
    return f(dest_flat, val_flat)


_GROWS = NP // 32
_GCH = 4
_GCROWS = _GROWS // _GCH


def _gather_body(tok_hbm, x_hbm, xs_hbm, idxv, rowsv0, rowsv1, sem0, sem1):
    wid = lax.axis_index("s") * 2 + lax.axis_index("c")
    base = wid * _GROWS
    pltpu.sync_copy(tok_hbm.at[pl.ds(base, _GROWS)], idxv)
    bufs = [(rowsv0, sem0), (rowsv1, sem1)]
    cps = [None, None]
    cps[0] = pltpu.async_copy(
        x_hbm.at[idxv.at[pl.ds(0, _GCROWS)]], rowsv0, sem0)
    for ch in range(1, _GCH):
        buf, sem = bufs[ch % 2]
        pbuf, _ = bufs[(ch - 1) % 2]
        cps[ch % 2] = pltpu.async_copy(
            x_hbm.at[idxv.at[pl.ds(ch * _GCROWS, _GCROWS)]], buf, sem)
        cps[(ch - 1) % 2].wait()
        pltpu.sync_copy(pbuf, xs_hbm.at[pl.ds(base + (ch - 1) * _GCROWS,
                                              _GCROWS)])
    cps[(_GCH - 1) % 2].wait()
    pltpu.sync_copy(bufs[(_GCH - 1) % 2][0],
                    xs_hbm.at[pl.ds(base + (_GCH - 1) * _GCROWS, _GCROWS)])


def _gather_rows(tok_s, xf):
    f = pl.kernel(
        _gather_body, mesh=plsc.VectorSubcoreMesh(**_SC_MESH),
        out_type=jax.ShapeDtypeStruct((NP, D), jnp.float32),
        scratch_types=[
            pltpu.VMEM((_GROWS,), jnp.int32),
            pltpu.VMEM((_GCROWS, D), jnp.float32),
            pltpu.VMEM((_GCROWS, D), jnp.float32),
            pltpu.SemaphoreType.DMA,
            pltpu.SemaphoreType.DMA,
        ],
    )
    return f(tok_s, xf)


def _ffn_body(te_ref, xs_ref, w1_ref, b1_ref, w2_ref, b2_ref, g_ref,
              eo_ref, acc_ref):
    h = pl.program_id(0)
    i = pl.program_id(1)
    rows = pl.ds(i * T_BLK, T_BLK)

    xb = xs_ref[...]
    hb = jnp.dot(xb, w1_ref[0], preferred_element_type=jnp.float32)
    hb = hb + b1_ref[0]
    hb = hb * 0.5 * (1.0 + lax.erf(hb * 0.7071067811865476))
    contrib = jnp.dot(hb, w2_ref[0], preferred_element_type=jnp.float32)

    @pl.when(h == 0)
    def _():
        acc_ref[rows, :] = contrib + b2_ref[0]

    @pl.when(h != 0)
    def _():
        eo_ref[...] = g_ref[0] * (acc_ref[rows, :] + contrib)


def _ffn(tile_e, xs, W1, b1r, W2, b2r, g3):
    grid_spec = pltpu.PrefetchScalarGridSpec(
        num_scalar_prefetch=1,
        grid=(N_H, NT),
        in_specs=[
            pl.BlockSpec((T_BLK, D), lambda h, i, te: (i, 0)),
            pl.BlockSpec((1, D, H_BLK), lambda h, i, te: (te[i], 0, h)),
            pl.BlockSpec((1, 1, H_BLK), lambda h, i, te: (te[i], 0, h)),
            pl.BlockSpec((1, H_BLK, D), lambda h, i, te: (te[i], h, 0)),
            pl.BlockSpec((1, 1, D), lambda h, i, te: (te[i], 0, 0)),
            pl.BlockSpec((1, T_BLK, 1), lambda h, i, te: (i, 0, 0)),
        ],
        out_specs=pl.BlockSpec((T_BLK, D), lambda h, i, te: (i, 0)),
        scratch_shapes=[pltpu.VMEM((NP, D), jnp.float32)],
    )
    return pl.pallas_call(
        _ffn_body,
        grid_spec=grid_spec,
        out_shape=jax.ShapeDtypeStruct((NP, D), jnp.float32),
    )(tile_e, xs, W1, b1r, W2, b2r, g3)


_CTOK = S // 32
_CCH = 4
_CCTOK = _CTOK // _CCH


def _combine_body(dest_hbm, eo_hbm, out_hbm, idxv, rowsv0, rowsv1, outv,
                  sem0, sem1):
    wid = lax.axis_index("s") * 2 + lax.axis_index("c")
    t00 = wid * _CTOK
    pltpu.sync_copy(dest_hbm.at[pl.ds(2 * t00, 2 * _CTOK)], idxv)
    bufs = [(rowsv0, sem0), (rowsv1, sem1)]
    cps = [None, None]
    cps[0] = pltpu.async_copy(
        eo_hbm.at[idxv.at[pl.ds(0, 2 * _CCTOK)]], rowsv0, sem0)
    for ch in range(_CCH):
        if ch + 1 < _CCH:
            buf, sem = bufs[(ch + 1) % 2]
            cps[(ch + 1) % 2] = pltpu.async_copy(
                eo_hbm.at[idxv.at[pl.ds(2 * (ch + 1) * _CCTOK, 2 * _CCTOK)]],
                buf, sem)
        rows, _ = bufs[ch % 2]
        cps[ch % 2].wait()

        def tok_body(i, _):
            for v in range(D // 16):
                sl = pl.ds(v * 16, 16)
                outv[i, sl] = rows[2 * i, sl] + rows[2 * i + 1, sl]
            return _
        lax.fori_loop(0, _CCTOK, tok_body, 0)
        pltpu.sync_copy(outv, out_hbm.at[pl.ds(t00 + ch * _CCTOK, _CCTOK)])


def _combine(dest_flat, eo):
    f = pl.kernel(
        _combine_body, mesh=plsc.VectorSubcoreMesh(**_SC_MESH),
        out_type=jax.ShapeDtypeStruct((S, D), jnp.float32),
        scratch_types=[
            pltpu.VMEM((2 * _CTOK,), jnp.int32),
            pltpu.VMEM((2 * _CCTOK, D), jnp.float32),
            pltpu.VMEM((2 * _CCTOK, D), jnp.float32),
            pltpu.VMEM((_CCTOK, D), jnp.float32),
            pltpu.SemaphoreType.DMA,
            pltpu.SemaphoreType.DMA,
        ],
    )
    return f(dest_flat, eo)


def kernel(x, Wg, bg, W1, b1, W2, b2):
    b, s, d = x.shape
    xf = x.reshape(s, d)

    dest2, val2, te = _route(xf, Wg, bg)
    dest_flat = dest2.reshape(NA)
    val_flat = val2.reshape(NA)

    tok_s, g_s = _scatter_slots(dest_flat, val_flat)
    xs = _gather_rows(tok_s, xf)

    eo = _ffn(te.reshape(T_BLK), xs, W1, b1.reshape(N_EXP, 1, HID),
              W2, b2.reshape(N_EXP, 1, D), g_s.reshape(NT, T_BLK, 1))

    out = _combine(dest_flat, eo)
    return out.reshape(b, s, d)
